# baseline (speedup 1.0000x reference)
import functools

import jax
import jax.numpy as jnp
from jax import lax
from jax.experimental import pallas as pl
from jax.experimental.pallas import tpu as pltpu
from jax.experimental.pallas import tpu_sc as plsc

L = 16
NW = 32
D = 2


def _ceil_div(a, b):
    return -(-a // b)


@functools.lru_cache(maxsize=None)
def _build_sc_normalize(ncol, nb, nr, n_max, n_std):
    cpp = 16
    cl = _ceil_div(_ceil_div(nr, cpp), 128) * 128
    n_full = nr // cl
    tail0 = n_full * cl
    tail_l = nr - tail0
    assert tail_l > 0 and tail_l % L == 0 and tail0 % 128 == 0
    cpp = n_full + 1
    units = ncol * cpp
    assert units % NW == 0
    upw = units // NW
    assert upw >= D
    n_max_pad = _ceil_div(n_max, L) * L
    n_std_pad = _ceil_div(n_std, L) * L
    ncol_pad = _ceil_div(ncol, L) * L

    mesh = plsc.VectorSubcoreMesh(core_axis_name="c", subcore_axis_name="s")

    @functools.partial(
        pl.kernel,
        mesh=mesh,
        out_type=jax.ShapeDtypeStruct((ncol, nb, nr), jnp.float32),
        compiler_params=pltpu.CompilerParams(
            needs_layout_passes=False, use_tc_tiling_on_sc=True),
        scratch_types=[
            pltpu.VMEM((ncol_pad,), jnp.float32),
            pltpu.VMEM((ncol_pad,), jnp.float32),
            pltpu.VMEM((n_max_pad,), jnp.float32),
            pltpu.VMEM((n_std_pad,), jnp.float32),
            pltpu.VMEM((n_std_pad,), jnp.float32),
            pltpu.VMEM((n_max_pad,), jnp.int32),
            pltpu.VMEM((n_std_pad,), jnp.int32),
            pltpu.VMEM((D, nb, cl), jnp.float32),
            pltpu.VMEM((D, nb, cl), jnp.float32),
            pltpu.VMEM((D, nb, tail_l), jnp.float32),
            pltpu.VMEM((D, nb, tail_l), jnp.float32),
        ] + [pltpu.SemaphoreType.DMA] * (2 * D + 1),
    )
    def normalize(x_hbm, maxn_hbm, mu_hbm, sd_hbm, maxidx_hbm, stdidx_hbm,
                  out_hbm,
                  a_tab, b_tab, maxn_v, mu_v, sd_v, maxidx_v, stdidx_v,
                  in_bufs, out_bufs, in_tails, out_tails,
                  *sems):
        sem_in = sems[:D]
        sem_out = sems[D:2 * D]
        sem_p = sems[2 * D]
        wid = lax.axis_index("s") * 2 + lax.axis_index("c")
        iota = lax.broadcasted_iota(jnp.int32, (L,), 0)

        def unit(i):
            u = i * NW + wid
            return u // cpp, u % cpp

        def _in_copies(i, j):
            c, sub = unit(i)
            l0 = pl.multiple_of(
                jnp.clip(sub, 0, n_full - 1) * cl, 128)
            full = pltpu.make_async_copy(
                x_hbm.at[c, :, pl.ds(l0, cl)], in_bufs.at[j], sem_in[j])
            tail = pltpu.make_async_copy(
                x_hbm.at[c, :, pl.ds(tail0, tail_l)], in_tails.at[j],
                sem_in[j])
            return full, tail, sub

        def _out_copies(i, j):
            c, sub = unit(i)
            l0 = pl.multiple_of(
                jnp.clip(sub, 0, n_full - 1) * cl, 128)
            full = pltpu.make_async_copy(
                out_bufs.at[j], out_hbm.at[c, :, pl.ds(l0, cl)], sem_out[j])
            tail = pltpu.make_async_copy(
                out_tails.at[j], out_hbm.at[c, :, pl.ds(tail0, tail_l)],
                sem_out[j])
            return full, tail, sub

        def start_in(i, j):
            full, tail, sub = _in_copies(i, j)

            @pl.when(sub < n_full)
            def _():
                full.start()

            @pl.when(sub == n_full)
            def _():
                tail.start()

        def wait_in(i, j):
            full, tail, sub = _in_copies(i, j)

            @pl.when(sub < n_full)
            def _():
                full.wait()

            @pl.when(sub == n_full)
            def _():
                tail.wait()

        def start_out(i, j):
            full, tail, sub = _out_copies(i, j)

            @pl.when(sub < n_full)
            def _():
                full.start()

            @pl.when(sub == n_full)
            def _():
                tail.start()

        def wait_out(i, j):
            full, tail, sub = _out_copies(i, j)

            @pl.when(sub < n_full)
            def _():
                full.wait()

            @pl.when(sub == n_full)
            def _():
                tail.wait()

        def compute(i, j):
            c, sub = unit(i)
            c_vec = jnp.full((L,), 0, jnp.int32) + c
            av = plsc.load_gather(a_tab, [c_vec])
            bv = plsc.load_gather(b_tab, [c_vec])
            ib = in_bufs.at[j]
            ob = out_bufs.at[j]
            ibt = in_tails.at[j]
            obt = out_tails.at[j]

            @pl.when(sub < n_full)
            def _():
                def tbody(t, carry):
                    l0 = pl.multiple_of(t * 128, 128)
                    for s in range(nb):
                        for k in range(128 // L):
                            o = l0 + k * L
                            ob[s, pl.ds(o, L)] = ib[s, pl.ds(o, L)] * av + bv
                    return carry
                lax.fori_loop(0, cl // 128, tbody, 0)

            @pl.when(sub == n_full)
            def _():
                def vbody(t, carry):
                    o = pl.multiple_of(t * L, L)
                    for s in range(nb):
                        obt[s, pl.ds(o, L)] = ibt[s, pl.ds(o, L)] * av + bv
                    return carry
                lax.fori_loop(0, tail_l // L, vbody, 0)

        for j in range(D):
            start_in(j, j)

        stage = [
            pltpu.make_async_copy(maxn_hbm, maxn_v, sem_p),
            pltpu.make_async_copy(mu_hbm, mu_v, sem_p),
            pltpu.make_async_copy(sd_hbm, sd_v, sem_p),
            pltpu.make_async_copy(maxidx_hbm, maxidx_v, sem_p),
            pltpu.make_async_copy(stdidx_hbm, stdidx_v, sem_p),
        ]
        for cp in stage:
            cp.start()
        for cp in stage:
            cp.wait()

        ones = jnp.full((L,), 1.0, jnp.float32)
        zeros = jnp.full((L,), 0.0, jnp.float32)
        for k in range(ncol_pad // L):
            a_tab[pl.ds(k * L, L)] = ones
            b_tab[pl.ds(k * L, L)] = zeros
        for j in range(n_max_pad // L):
            idx = maxidx_v[pl.ds(j * L, L)]
            inv = ones / maxn_v[pl.ds(j * L, L)]
            m = (iota + j * L) < n_max
            plsc.store_scatter(a_tab, [idx], inv, mask=m)
        for j in range(n_std_pad // L):
            idx = stdidx_v[pl.ds(j * L, L)]
            inv = ones / sd_v[pl.ds(j * L, L)]
            mb = (zeros - mu_v[pl.ds(j * L, L)]) * inv
            m = (iota + j * L) < n_std
            plsc.store_scatter(a_tab, [idx], inv, mask=m)
            plsc.store_scatter(b_tab, [idx], mb, mask=m)

        def lbody(blk, carry):
            for j in range(D):
                u = blk * D + j

                @pl.when(u < upw)
                def _():
                    wait_in(u, j)

                    @pl.when(blk > 0)
                    def _():
                        wait_out(u - D, j)
                    compute(u, j)
                    start_out(u, j)

                    @pl.when(u + D < upw)
                    def _():
                        start_in(u + D, j)
            return carry

        lax.fori_loop(0, _ceil_div(upw, D), lbody, 0)
        for j in range(D):
            wait_out(((upw - 1 - j) // D) * D + j, j)

    return normalize


def kernel(x, max_norm, std_norm_mu, std_norm_sd, max_norm_idx, std_norm_idx):
    ncol = x.shape[-1]
    nb = x.shape[0]
    nr = x.shape[1]
    n_max = max_norm.shape[0]
    n_std = std_norm_idx.shape[0]
    n_max_pad = _ceil_div(n_max, L) * L
    n_std_pad = _ceil_div(n_std, L) * L

    def _pad(v, n_pad, fill, dtype):
        pad = jnp.full((n_pad - v.shape[0],), fill, dtype)
        return jnp.concatenate([v.astype(dtype), pad])

    maxn = _pad(max_norm, n_max_pad, 1.0, jnp.float32)
    mu = _pad(std_norm_mu, n_std_pad, 0.0, jnp.float32)
    sd = _pad(std_norm_sd, n_std_pad, 1.0, jnp.float32)
    maxidx = _pad(max_norm_idx, n_max_pad, 0, jnp.int32)
    stdidx = _pad(std_norm_idx, n_std_pad, 0, jnp.int32)
    fn = _build_sc_normalize(ncol, nb, nr, n_max, n_std)
    xt = jnp.transpose(x, (2, 0, 1))
    out_t = fn(xt, maxn, mu, sd, maxidx, stdidx)
    return jnp.transpose(out_t, (1, 2, 0))


# speedup vs baseline: 1.2502x; 1.2502x over previous
import functools

import jax
import jax.numpy as jnp
from jax import lax
from jax.experimental import pallas as pl
from jax.experimental.pallas import tpu as pltpu
from jax.experimental.pallas import tpu_sc as plsc

L = 16
NW = 32
D = 4


def _ceil_div(a, b):
    return -(-a // b)


@functools.lru_cache(maxsize=None)
def _build_sc_normalize(ncol, nb, nr, n_max, n_std):
    cpp = 32
    cl = _ceil_div(_ceil_div(nr, cpp), 128) * 128
    n_full = nr // cl
    tail0 = n_full * cl
    tail_l = nr - tail0
    assert tail_l > 0 and tail_l % L == 0 and tail0 % 128 == 0
    cpp = n_full + 1
    units = ncol * cpp
    assert units % NW == 0
    upw = units // NW
    assert upw >= D
    n_max_pad = _ceil_div(n_max, L) * L
    n_std_pad = _ceil_div(n_std, L) * L
    ncol_pad = _ceil_div(ncol, L) * L

    mesh = plsc.VectorSubcoreMesh(core_axis_name="c", subcore_axis_name="s")

    @functools.partial(
        pl.kernel,
        mesh=mesh,
        out_type=jax.ShapeDtypeStruct((ncol, nb, nr), jnp.float32),
        compiler_params=pltpu.CompilerParams(
            needs_layout_passes=False, use_tc_tiling_on_sc=True),
        scratch_types=[
            pltpu.VMEM((ncol_pad,), jnp.float32),
            pltpu.VMEM((ncol_pad,), jnp.float32),
            pltpu.VMEM((n_max_pad,), jnp.float32),
            pltpu.VMEM((n_std_pad,), jnp.float32),
            pltpu.VMEM((n_std_pad,), jnp.float32),
            pltpu.VMEM((n_max_pad,), jnp.int32),
            pltpu.VMEM((n_std_pad,), jnp.int32),
            pltpu.VMEM((D, nb, cl), jnp.float32),
            pltpu.VMEM((D, nb, cl), jnp.float32),
            pltpu.VMEM((D, nb, tail_l), jnp.float32),
            pltpu.VMEM((D, nb, tail_l), jnp.float32),
        ] + [pltpu.SemaphoreType.DMA] * (2 * D + 1),
    )
    def normalize(x_hbm, maxn_hbm, mu_hbm, sd_hbm, maxidx_hbm, stdidx_hbm,
                  out_hbm,
                  a_tab, b_tab, maxn_v, mu_v, sd_v, maxidx_v, stdidx_v,
                  in_bufs, out_bufs, in_tails, out_tails,
                  *sems):
        sem_in = sems[:D]
        sem_out = sems[D:2 * D]
        sem_p = sems[2 * D]
        wid = lax.axis_index("s") * 2 + lax.axis_index("c")
        iota = lax.broadcasted_iota(jnp.int32, (L,), 0)

        def unit(i):
            u = i * NW + wid
            return u // cpp, u % cpp

        def _in_copies(i, j):
            c, sub = unit(i)
            l0 = pl.multiple_of(
                jnp.clip(sub, 0, n_full - 1) * cl, 128)
            full = pltpu.make_async_copy(
                x_hbm.at[c, :, pl.ds(l0, cl)], in_bufs.at[j], sem_in[j])
            tail = pltpu.make_async_copy(
                x_hbm.at[c, :, pl.ds(tail0, tail_l)], in_tails.at[j],
                sem_in[j])
            return full, tail, sub

        def _out_copies(i, j):
            c, sub = unit(i)
            l0 = pl.multiple_of(
                jnp.clip(sub, 0, n_full - 1) * cl, 128)
            full = pltpu.make_async_copy(
                out_bufs.at[j], out_hbm.at[c, :, pl.ds(l0, cl)], sem_out[j])
            tail = pltpu.make_async_copy(
                out_tails.at[j], out_hbm.at[c, :, pl.ds(tail0, tail_l)],
                sem_out[j])
            return full, tail, sub

        def start_in(i, j):
            full, tail, sub = _in_copies(i, j)

            @pl.when(sub < n_full)
            def _():
                full.start()

            @pl.when(sub == n_full)
            def _():
                tail.start()

        def wait_in(i, j):
            full, tail, sub = _in_copies(i, j)

            @pl.when(sub < n_full)
            def _():
                full.wait()

            @pl.when(sub == n_full)
            def _():
                tail.wait()

        def start_out(i, j):
            full, tail, sub = _out_copies(i, j)

            @pl.when(sub < n_full)
            def _():
                full.start()

            @pl.when(sub == n_full)
            def _():
                tail.start()

        def wait_out(i, j):
            full, tail, sub = _out_copies(i, j)

            @pl.when(sub < n_full)
            def _():
                full.wait()

            @pl.when(sub == n_full)
            def _():
                tail.wait()

        def compute(i, j):
            c, sub = unit(i)
            c_vec = jnp.full((L,), 0, jnp.int32) + c
            av = plsc.load_gather(a_tab, [c_vec])
            bv = plsc.load_gather(b_tab, [c_vec])
            ib = in_bufs.at[j]
            ob = out_bufs.at[j]
            ibt = in_tails.at[j]
            obt = out_tails.at[j]

            @pl.when(sub < n_full)
            def _():
                def tbody(t, carry):
                    l0 = pl.multiple_of(t * 128, 128)
                    for s in range(nb):
                        for k in range(128 // L):
                            o = l0 + k * L
                            ob[s, pl.ds(o, L)] = ib[s, pl.ds(o, L)] * av + bv
                    return carry
                lax.fori_loop(0, cl // 128, tbody, 0)

            @pl.when(sub == n_full)
            def _():
                def vbody(t, carry):
                    o = pl.multiple_of(t * L, L)
                    for s in range(nb):
                        obt[s, pl.ds(o, L)] = ibt[s, pl.ds(o, L)] * av + bv
                    return carry
                lax.fori_loop(0, tail_l // L, vbody, 0)

        for j in range(D):
            start_in(j, j)

        stage = [
            pltpu.make_async_copy(maxn_hbm, maxn_v, sem_p),
            pltpu.make_async_copy(mu_hbm, mu_v, sem_p),
            pltpu.make_async_copy(sd_hbm, sd_v, sem_p),
            pltpu.make_async_copy(maxidx_hbm, maxidx_v, sem_p),
            pltpu.make_async_copy(stdidx_hbm, stdidx_v, sem_p),
        ]
        for cp in stage:
            cp.start()
        for cp in stage:
            cp.wait()

        ones = jnp.full((L,), 1.0, jnp.float32)
        zeros = jnp.full((L,), 0.0, jnp.float32)
        for k in range(ncol_pad // L):
            a_tab[pl.ds(k * L, L)] = ones
            b_tab[pl.ds(k * L, L)] = zeros
        for j in range(n_max_pad // L):
            idx = maxidx_v[pl.ds(j * L, L)]
            inv = ones / maxn_v[pl.ds(j * L, L)]
            m = (iota + j * L) < n_max
            plsc.store_scatter(a_tab, [idx], inv, mask=m)
        for j in range(n_std_pad // L):
            idx = stdidx_v[pl.ds(j * L, L)]
            inv = ones / sd_v[pl.ds(j * L, L)]
            mb = (zeros - mu_v[pl.ds(j * L, L)]) * inv
            m = (iota + j * L) < n_std
            plsc.store_scatter(a_tab, [idx], inv, mask=m)
            plsc.store_scatter(b_tab, [idx], mb, mask=m)

        def lbody(blk, carry):
            for j in range(D):
                u = blk * D + j

                @pl.when(u < upw)
                def _():
                    wait_in(u, j)

                    @pl.when(blk > 0)
                    def _():
                        wait_out(u - D, j)
                    compute(u, j)
                    start_out(u, j)

                    @pl.when(u + D < upw)
                    def _():
                        start_in(u + D, j)
            return carry

        lax.fori_loop(0, _ceil_div(upw, D), lbody, 0)
        for j in range(D):
            wait_out(((upw - 1 - j) // D) * D + j, j)

    return normalize


def kernel(x, max_norm, std_norm_mu, std_norm_sd, max_norm_idx, std_norm_idx):
    ncol = x.shape[-1]
    nb = x.shape[0]
    nr = x.shape[1]
    n_max = max_norm.shape[0]
    n_std = std_norm_idx.shape[0]
    n_max_pad = _ceil_div(n_max, L) * L
    n_std_pad = _ceil_div(n_std, L) * L

    def _pad(v, n_pad, fill, dtype):
        pad = jnp.full((n_pad - v.shape[0],), fill, dtype)
        return jnp.concatenate([v.astype(dtype), pad])

    maxn = _pad(max_norm, n_max_pad, 1.0, jnp.float32)
    mu = _pad(std_norm_mu, n_std_pad, 0.0, jnp.float32)
    sd = _pad(std_norm_sd, n_std_pad, 1.0, jnp.float32)
    maxidx = _pad(max_norm_idx, n_max_pad, 0, jnp.int32)
    stdidx = _pad(std_norm_idx, n_std_pad, 0, jnp.int32)
    fn = _build_sc_normalize(ncol, nb, nr, n_max, n_std)
    xt = jnp.transpose(x, (2, 0, 1))
    out_t = fn(xt, maxn, mu, sd, maxidx, stdidx)
    return jnp.transpose(out_t, (1, 2, 0))
